# baseline (speedup 1.0000x reference)
import functools

import jax
import jax.numpy as jnp
from jax.experimental import pallas as pl

_CH = 200


def _reduce_body(x_ref, o_ref, *, scale):
    s = pl.program_id(1)
    partial = jnp.sum(x_ref[...], axis=1)[:, None, :] * scale

    @pl.when(s == 0)
    def _init():
        o_ref[...] = partial

    @pl.when(s != 0)
    def _acc():
        o_ref[...] += partial


def kernel(x, protos):
    N, M, C, T, V = x.shape
    P = protos.shape[0]
    scale = 1.0 / (M * P)

    xt = jnp.transpose(x, (0, 1, 4, 3, 2)).reshape(N, M * V * T, C)
    rows = M * V * T
    steps = rows // _CH

    out = pl.pallas_call(
        functools.partial(_reduce_body, scale=scale),
        grid=(N, steps),
        in_specs=[pl.BlockSpec((1, _CH, C), lambda n, s: (n, s, 0))],
        out_specs=pl.BlockSpec((1, 1, C), lambda n, s: (n, 0, 0)),
        out_shape=jax.ShapeDtypeStruct((N, 1, C), x.dtype),
    )(xt)
    return out.reshape(N, C)


# speedup vs baseline: 2.8536x; 2.8536x over previous
import functools

import jax
import jax.numpy as jnp
from jax.experimental import pallas as pl

_CH = 800


def _reduce_body(x_ref, o_ref, *, scale):
    s = pl.program_id(1)
    partial = jnp.sum(x_ref[...], axis=1)[:, None, :] * scale

    @pl.when(s == 0)
    def _init():
        o_ref[...] = partial

    @pl.when(s != 0)
    def _acc():
        o_ref[...] += partial


def kernel(x, protos):
    N, M, C, T, V = x.shape
    P = protos.shape[0]
    scale = 1.0 / (M * P)

    xt = jnp.transpose(x, (0, 1, 4, 3, 2)).reshape(N, M * V * T, C)
    rows = M * V * T
    steps = rows // _CH

    out = pl.pallas_call(
        functools.partial(_reduce_body, scale=scale),
        grid=(N, steps),
        in_specs=[pl.BlockSpec((1, _CH, C), lambda n, s: (n, s, 0))],
        out_specs=pl.BlockSpec((1, 1, C), lambda n, s: (n, 0, 0)),
        out_shape=jax.ShapeDtypeStruct((N, 1, C), x.dtype),
    )(xt)
    return out.reshape(N, C)


# speedup vs baseline: 4.2964x; 1.5056x over previous
import functools

import jax
import jax.numpy as jnp
from jax.experimental import pallas as pl

_CH = 1600


def _reduce_body(x_ref, o_ref, *, scale):
    s = pl.program_id(1)
    partial = jnp.sum(x_ref[...], axis=1)[:, None, :] * scale

    @pl.when(s == 0)
    def _init():
        o_ref[...] = partial

    @pl.when(s != 0)
    def _acc():
        o_ref[...] += partial


def kernel(x, protos):
    N, M, C, T, V = x.shape
    P = protos.shape[0]
    scale = 1.0 / (M * P)

    xt = jnp.transpose(x, (0, 1, 4, 3, 2)).reshape(N, M * V * T, C)
    rows = M * V * T
    steps = rows // _CH

    out = pl.pallas_call(
        functools.partial(_reduce_body, scale=scale),
        grid=(N, steps),
        in_specs=[pl.BlockSpec((1, _CH, C), lambda n, s: (n, s, 0))],
        out_specs=pl.BlockSpec((1, 1, C), lambda n, s: (n, 0, 0)),
        out_shape=jax.ShapeDtypeStruct((N, 1, C), x.dtype),
    )(xt)
    return out.reshape(N, C)


# speedup vs baseline: 5.8392x; 1.3591x over previous
import functools

import jax
import jax.numpy as jnp
from jax.experimental import pallas as pl

_CH = 3200


def _reduce_body(x_ref, o_ref, *, scale):
    s = pl.program_id(1)
    partial = jnp.sum(x_ref[...], axis=1)[:, None, :] * scale

    @pl.when(s == 0)
    def _init():
        o_ref[...] = partial

    @pl.when(s != 0)
    def _acc():
        o_ref[...] += partial


def kernel(x, protos):
    N, M, C, T, V = x.shape
    P = protos.shape[0]
    scale = 1.0 / (M * P)

    xt = jnp.transpose(x, (0, 1, 4, 3, 2)).reshape(N, M * V * T, C)
    rows = M * V * T
    steps = rows // _CH

    out = pl.pallas_call(
        functools.partial(_reduce_body, scale=scale),
        grid=(N, steps),
        in_specs=[pl.BlockSpec((1, _CH, C), lambda n, s: (n, s, 0))],
        out_specs=pl.BlockSpec((1, 1, C), lambda n, s: (n, 0, 0)),
        out_shape=jax.ShapeDtypeStruct((N, 1, C), x.dtype),
    )(xt)
    return out.reshape(N, C)


# speedup vs baseline: 6.9153x; 1.1843x over previous
import functools

import jax
import jax.numpy as jnp
from jax.experimental import pallas as pl

_NB = 2


def _reduce_body(x_ref, o_ref, *, scale):
    o_ref[...] = jnp.sum(x_ref[...], axis=1, keepdims=True) * scale


def kernel(x, protos):
    N, M, C, T, V = x.shape
    P = protos.shape[0]
    scale = 1.0 / (M * P)
    rows = M * V * T

    xt = jnp.transpose(x, (0, 1, 4, 3, 2)).reshape(N, rows, C)

    out = pl.pallas_call(
        functools.partial(_reduce_body, scale=scale),
        grid=(N // _NB,),
        in_specs=[pl.BlockSpec((_NB, rows, C), lambda i: (i, 0, 0))],
        out_specs=pl.BlockSpec((_NB, 1, C), lambda i: (i, 0, 0)),
        out_shape=jax.ShapeDtypeStruct((N, 1, C), x.dtype),
    )(xt)
    return out.reshape(N, C)


# speedup vs baseline: 7.0410x; 1.0182x over previous
import functools

import jax
import jax.numpy as jnp
from jax.experimental import pallas as pl

_NB = 4


def _reduce_body(x_ref, o_ref, *, scale):
    o_ref[...] = jnp.sum(x_ref[...], axis=1, keepdims=True) * scale


def kernel(x, protos):
    N, M, C, T, V = x.shape
    P = protos.shape[0]
    scale = 1.0 / (M * P)
    rows = M * V * T

    xt = jnp.transpose(x, (0, 1, 4, 3, 2)).reshape(N, rows, C)

    out = pl.pallas_call(
        functools.partial(_reduce_body, scale=scale),
        grid=(N // _NB,),
        in_specs=[pl.BlockSpec((_NB, rows, C), lambda i: (i, 0, 0))],
        out_specs=pl.BlockSpec((_NB, 1, C), lambda i: (i, 0, 0)),
        out_shape=jax.ShapeDtypeStruct((N, 1, C), x.dtype),
    )(xt)
    return out.reshape(N, C)
